# Initial kernel scaffold; baseline (speedup 1.0000x reference)
import functools

import jax
import jax.numpy as jnp
from jax import lax
from jax.experimental import pallas as pl
from jax.experimental.pallas import tpu as pltpu
from jax.experimental.pallas import tpu_sc as plsc

_B, _C, _N = 4, 64, 32768
_M, _K = 8192, 16
_NC, _NS, _L = 2, 16, 16
_NW = _NC * _NS
_ROWS = _B * _M
_RPW = _ROWS // _NW
_R = 32
_STEPS = _RPW // _R
_IDXW = 128
_G = (_R * _K) // _IDXW


def _gather_max(table, idx):
  mesh = plsc.VectorSubcoreMesh(
      core_axis_name="c", subcore_axis_name="s",
      num_cores=_NC, num_subcores=_NS)

  @functools.partial(
      pl.kernel,
      out_type=jax.ShapeDtypeStruct((_ROWS, _C), jnp.float32),
      mesh=mesh,
      scratch_types=[
          pltpu.VMEM((_G, _IDXW), jnp.int32),
          pltpu.VMEM((_R * _K, _C), jnp.float32),
          pltpu.VMEM((_R, _C), jnp.float32),
          pltpu.SemaphoreType.DMA,
      ],
  )
  def body(table_hbm, idx_hbm, out_hbm, idx_v, rows_v, out_v, sem):
    wid = lax.axis_index("s") * _NC + lax.axis_index("c")
    wbase = wid * _RPW

    @pl.loop(0, _STEPS)
    def _step(s):
      base = wbase + s * _R
      pltpu.sync_copy(idx_hbm.at[pl.ds(base * _K // _IDXW, _G)], idx_v)
      copies = [
          pltpu.async_copy(
              table_hbm.at[idx_v.at[g]],
              rows_v.at[pl.ds(g * _IDXW, _IDXW)],
              sem)
          for g in range(_G)
      ]
      for c in copies:
        c.wait()

      @pl.loop(0, _R)
      def _row(r):
        rb = r * _K
        for j in range(_C // _L):
          sl = pl.ds(j * _L, _L)
          acc = rows_v[rb, sl]
          for kk in range(1, _K):
            acc = jnp.maximum(acc, rows_v[rb + kk, sl])
          out_v[r, sl] = acc

      pltpu.sync_copy(out_v, out_hbm.at[pl.ds(base, _R)])

  return body(table, idx)


def kernel(input, points, support_points, indices):
  table = input.transpose(0, 2, 1).reshape(_B * _N, _C)
  offs = (jnp.arange(_B, dtype=jnp.int32) * _N).reshape(_B, 1, 1)
  idx = (indices.astype(jnp.int32) + offs).reshape(_ROWS * _K // _IDXW, _IDXW)
  out = _gather_max(table, idx)
  features = out.reshape(_B, _M, _C).transpose(0, 2, 1)
  return (features, support_points, indices)


# speedup vs baseline: 2450.2432x; 2450.2432x over previous
import functools

import jax
import jax.numpy as jnp
from jax import lax
from jax.experimental import pallas as pl
from jax.experimental.pallas import tpu as pltpu
from jax.experimental.pallas import tpu_sc as plsc

_B, _C, _N = 4, 64, 32768
_M, _K = 8192, 16
_NC, _NS, _L = 2, 16, 16
_NW = _NC * _NS
_ROWS = _B * _M
_RPW = _ROWS // _NW
_R = 64
_STEPS = _RPW // _R
_IDXW = 128
_G = (_R * _K) // _IDXW


def _gather_max(table, idx):
  mesh = plsc.VectorSubcoreMesh(
      core_axis_name="c", subcore_axis_name="s",
      num_cores=_NC, num_subcores=_NS)

  @functools.partial(
      pl.kernel,
      out_type=jax.ShapeDtypeStruct((_ROWS, _C), jnp.float32),
      mesh=mesh,
      scratch_types=[
          pltpu.VMEM((_G, _IDXW), jnp.int32),
          pltpu.VMEM((_R * _K, _C), jnp.float32),
          pltpu.VMEM((_R, _C), jnp.float32),
          pltpu.SemaphoreType.DMA,
      ],
      compiler_params=pltpu.CompilerParams(use_tc_tiling_on_sc=False),
  )
  def body(table_hbm, idx_hbm, out_hbm, idx_v, rows_v, out_v, sem):
    wid = lax.axis_index("s") * _NC + lax.axis_index("c")
    wbase = wid * _RPW

    @pl.loop(0, _STEPS)
    def _step(s):
      base = pl.multiple_of(wbase + s * _R, _R)
      pltpu.sync_copy(
          idx_hbm.at[pl.ds(pl.multiple_of(base * _K // _IDXW, 8), _G)], idx_v)
      copies = [
          pltpu.async_copy(
              table_hbm.at[idx_v.at[g]],
              rows_v.at[pl.ds(g * _IDXW, _IDXW)],
              sem)
          for g in range(_G)
      ]
      for c in copies:
        c.wait()

      @pl.loop(0, _R)
      def _row(r):
        rb = r * _K
        for j in range(_C // _L):
          sl = pl.ds(j * _L, _L)
          acc = rows_v[rb, sl]
          for kk in range(1, _K):
            acc = jnp.maximum(acc, rows_v[rb + kk, sl])
          out_v[r, sl] = acc

      pltpu.sync_copy(out_v, out_hbm.at[pl.ds(base, _R)])

  return body(table, idx)


def kernel(input, points, support_points, indices):
  table = input.transpose(0, 2, 1).reshape(_B * _N, _C)
  offs = (jnp.arange(_B, dtype=jnp.int32) * _N).reshape(_B, 1, 1)
  idx = (indices.astype(jnp.int32) + offs).reshape(_ROWS * _K // _IDXW, _IDXW)
  out = _gather_max(table, idx)
  features = out.reshape(_B, _M, _C).transpose(0, 2, 1)
  return (features, support_points, indices)


# speedup vs baseline: 2905.4154x; 1.1858x over previous
import functools

import jax
import jax.numpy as jnp
from jax import lax
from jax.experimental import pallas as pl
from jax.experimental.pallas import tpu as pltpu
from jax.experimental.pallas import tpu_sc as plsc

_B, _C, _N = 4, 64, 32768
_M, _K = 8192, 16
_NC, _NS, _L = 2, 16, 16
_NW = _NC * _NS
_ROWS = _B * _M
_RPW = _ROWS // _NW
_R = 32
_STEPS = _RPW // _R
_IDXW = 128
_G = (_R * _K) // _IDXW
_CB = _C // _L


def _gather_max(table, idx):
  mesh = plsc.VectorSubcoreMesh(
      core_axis_name="c", subcore_axis_name="s",
      num_cores=_NC, num_subcores=_NS)

  @functools.partial(
      pl.kernel,
      out_type=jax.ShapeDtypeStruct((_ROWS, _C), jnp.float32),
      mesh=mesh,
      scratch_types=[
          pltpu.VMEM((2, _G, _IDXW), jnp.int32),
          pltpu.VMEM((2, _R * _K, _C), jnp.float32),
          pltpu.VMEM((2, _R, _C), jnp.float32),
          pltpu.SemaphoreType.DMA,
          pltpu.SemaphoreType.DMA,
      ],
      compiler_params=pltpu.CompilerParams(use_tc_tiling_on_sc=False),
  )
  def body(table_hbm, idx_hbm, out_hbm, idx_v, rows_v, out_v, sg0, sg1):
    wid = lax.axis_index("s") * _NC + lax.axis_index("c")
    wbase = wid * _RPW
    cbase = wid * _STEPS
    sems = (sg0, sg1)

    def fire(s, P):
      pltpu.sync_copy(idx_hbm.at[cbase + s], idx_v.at[P])
      for g in range(_G):
        pltpu.async_copy(
            table_hbm.at[idx_v.at[P, g]],
            rows_v.at[P, pl.ds(g * _IDXW, _IDXW)],
            sems[P])

    def drain(P):
      for g in range(_G):
        pltpu.make_async_copy(
            table_hbm.at[idx_v.at[P, g]],
            rows_v.at[P, pl.ds(g * _IDXW, _IDXW)],
            sems[P]).wait()

    def compute(P):
      @pl.loop(0, _R)
      def _row(r):
        rb = r * _K
        for j in range(_CB):
          sl = pl.ds(j * _L, _L)
          acc = rows_v[P, rb, sl]
          for kk in range(1, _K):
            acc = jnp.maximum(acc, rows_v[P, rb + kk, sl])
          out_v[P, r, sl] = acc

    def store(s, P):
      base = pl.multiple_of(wbase + s * _R, _R)
      pltpu.sync_copy(out_v.at[P], out_hbm.at[pl.ds(base, _R)])

    fire(0, 0)

    @pl.loop(0, _STEPS // 2)
    def _pair(p):
      s0 = p * 2
      fire(s0 + 1, 1)
      drain(0)
      compute(0)
      store(s0, 0)

      @pl.when(p < _STEPS // 2 - 1)
      def _():
        fire(s0 + 2, 0)

      drain(1)
      compute(1)
      store(s0 + 1, 1)

  return body(table, idx)


def kernel(input, points, support_points, indices):
  table = input.transpose(0, 2, 1).reshape(_B * _N, _C)
  offs = (jnp.arange(_B, dtype=jnp.int32) * _N).reshape(_B, 1, 1)
  idx = (indices.astype(jnp.int32) + offs).reshape(_NW * _STEPS, _G, _IDXW)
  out = _gather_max(table, idx)
  features = out.reshape(_B, _M, _C).transpose(0, 2, 1)
  return (features, support_points, indices)


# speedup vs baseline: 3131.7314x; 1.0779x over previous
import functools

import jax
import jax.numpy as jnp
from jax import lax
from jax.experimental import pallas as pl
from jax.experimental.pallas import tpu as pltpu
from jax.experimental.pallas import tpu_sc as plsc

_B, _C, _N = 4, 64, 32768
_M, _K = 8192, 16
_NC, _NS, _L = 2, 16, 16
_NW = _NC * _NS
_ROWS = _B * _M
_RPW = _ROWS // _NW
_R = 32
_STEPS = _RPW // _R
_IDXW = 128
_G = (_R * _K) // _IDXW
_IDXR = _RPW * _K // _IDXW


def _gather_max(table, idx):
  mesh = plsc.VectorSubcoreMesh(
      core_axis_name="c", subcore_axis_name="s",
      num_cores=_NC, num_subcores=_NS)

  @functools.partial(
      pl.kernel,
      out_type=jax.ShapeDtypeStruct((_ROWS, _C), jnp.float32),
      mesh=mesh,
      scratch_types=[
          pltpu.VMEM((_IDXR, _IDXW), jnp.int32),
          pltpu.VMEM((2, _R * _K, _C), jnp.float32),
          pltpu.VMEM((2, _R, _C), jnp.float32),
          pltpu.SemaphoreType.DMA,
          pltpu.SemaphoreType.DMA,
          pltpu.SemaphoreType.DMA,
          pltpu.SemaphoreType.DMA,
      ],
      compiler_params=pltpu.CompilerParams(use_tc_tiling_on_sc=False),
  )
  def body(table_hbm, idx_hbm, out_hbm, idx_v, rows_v, out_v, sg0, sg1, so0, so1):
    wid = lax.axis_index("s") * _NC + lax.axis_index("c")
    wbase = wid * _RPW
    sg = (sg0, sg1)
    so = (so0, so1)

    pltpu.sync_copy(idx_hbm.at[wid], idx_v)

    def fire(s, P):
      for g in range(_G):
        pltpu.async_copy(
            table_hbm.at[idx_v.at[s * _G + g]],
            rows_v.at[P, pl.ds(g * _IDXW, _IDXW)],
            sg[P])

    def drain(s, P):
      for g in range(_G):
        pltpu.make_async_copy(
            table_hbm.at[idx_v.at[s * _G + g]],
            rows_v.at[P, pl.ds(g * _IDXW, _IDXW)],
            sg[P]).wait()

    def compute(P):
      @pl.loop(0, _R)
      def _row(r):
        rb = r * _K
        for j in range(_C // _L):
          sl = pl.ds(j * _L, _L)
          acc = rows_v[P, rb, sl]
          for kk in range(1, _K):
            acc = jnp.maximum(acc, rows_v[P, rb + kk, sl])
          out_v[P, r, sl] = acc

    def store(s, P):
      base = pl.multiple_of(wbase + s * _R, _R)
      pltpu.async_copy(out_v.at[P], out_hbm.at[pl.ds(base, _R)], so[P])

    def drain_store(s, P):
      base = pl.multiple_of(wbase + s * _R, _R)
      pltpu.make_async_copy(
          out_v.at[P], out_hbm.at[pl.ds(base, _R)], so[P]).wait()

    fire(0, 0)

    @pl.loop(0, _STEPS // 2)
    def _pair(p):
      s0 = p * 2
      fire(s0 + 1, 1)
      drain(s0, 0)

      @pl.when(p > 0)
      def _():
        drain_store(s0 - 2, 0)

      compute(0)
      store(s0, 0)

      @pl.when(p < _STEPS // 2 - 1)
      def _():
        fire(s0 + 2, 0)

      drain(s0 + 1, 1)

      @pl.when(p > 0)
      def _():
        drain_store(s0 - 1, 1)

      compute(1)
      store(s0 + 1, 1)

    drain_store(_STEPS - 2, 0)
    drain_store(_STEPS - 1, 1)

  return body(table, idx)


def kernel(input, points, support_points, indices):
  table = input.transpose(0, 2, 1).reshape(_B * _N, _C)
  offs = (jnp.arange(_B, dtype=jnp.int32) * _N).reshape(_B, 1, 1)
  idx = (indices.astype(jnp.int32) + offs).reshape(_NW, _IDXR, _IDXW)
  out = _gather_max(table, idx)
  features = out.reshape(_B, _M, _C).transpose(0, 2, 1)
  return (features, support_points, indices)
